# baseline (speedup 1.0000x reference)
import functools

import jax
import jax.numpy as jnp
from jax.experimental import pallas as pl
from jax.experimental.pallas import tpu as pltpu

_NUM_BOXES_PADDED = 128
_LEVELS = ((8, 200, 336), (8, 100, 168), (8, 50, 84))
_STEPS = 8


def _lamm_body(h0_ref, h1_ref, h2_ref, lab_ref, dims_ref, out_ref, acc_ref):
    i = pl.program_id(0)
    dimx = dims_ref[0, 0]
    dimy = dims_ref[0, 1]
    lab = lab_ref[:, :]

    for j, (h_ref, (_, hgt, wid)) in enumerate(
            zip((h0_ref, h1_ref, h2_ref), _LEVELS)):
        p = jnp.sum(h_ref[0, :, :])
        prev = jnp.where(i == 0, 0.0, acc_ref[0, j])
        acc_ref[0:1, j:j + 1] = jnp.reshape(prev + p, (1, 1))

    for j, (_, hgt, wid) in enumerate(_LEVELS):
        @pl.when(i == j)
        def _():
            sx = wid / dimx
            sy = hgt / dimy
            x1 = jnp.clip(jnp.round(lab[:, 0:1] * sx), 0.0, wid - 1.0)
            y1 = jnp.clip(jnp.round(lab[:, 1:2] * sy), 0.0, hgt - 1.0)
            x2 = jnp.clip(jnp.round(lab[:, 2:3] * sx), 0.0, float(wid))
            y2 = jnp.clip(jnp.round(lab[:, 3:4] * sy), 0.0, float(hgt))
            valid = ((x2 > x1) & (y2 > y1)).astype(jnp.float32)
            xx = jax.lax.broadcasted_iota(
                jnp.int32, (_NUM_BOXES_PADDED, wid), 1).astype(jnp.float32)
            yy = jax.lax.broadcasted_iota(
                jnp.int32, (_NUM_BOXES_PADDED, hgt), 1).astype(jnp.float32)
            xm = ((xx >= x1) & (xx < x2)).astype(jnp.float32) * valid
            ym = ((yy >= y1) & (yy < y2)).astype(jnp.float32)
            cov = jax.lax.dot_general(
                ym, xm, (((0,), (0,)), ((), ())),
                preferred_element_type=jnp.float32,
            )
            area = jnp.sum((cov > 0.5).astype(jnp.float32))
            acc_ref[0:1, 4 + j:5 + j] = jnp.reshape(area, (1, 1))

    @pl.when(i == _STEPS - 1)
    def _():
        total = jnp.float32(0.0)
        for j, (n, hgt, wid) in enumerate(_LEVELS):
            s = acc_ref[0, j]
            area = acc_ref[0, 4 + j]
            li = (s / float(n * hgt * wid) - area / float(hgt * wid)) ** 2
            total = total + li
        out_ref[:, :] = jnp.reshape(total / 3.0, (1, 1))


def kernel(h0, h1, h2, label, im_dimx, im_dimy):
    h0f = h0.reshape(8, 200, 336)
    h1f = h1.reshape(8, 100, 168)
    h2f = h2.reshape(8, 50, 84)
    lab = jnp.pad(label.astype(jnp.float32),
                  ((0, _NUM_BOXES_PADDED - label.shape[0]), (0, 0)))
    dims = jnp.stack([jnp.asarray(im_dimx, jnp.float32),
                      jnp.asarray(im_dimy, jnp.float32)]).reshape(1, 2)
    out = pl.pallas_call(
        _lamm_body,
        grid=(_STEPS,),
        in_specs=[
            pl.BlockSpec((1, 200, 336), lambda i: (i, 0, 0)),
            pl.BlockSpec((1, 100, 168), lambda i: (i, 0, 0)),
            pl.BlockSpec((1, 50, 84), lambda i: (i, 0, 0)),
            pl.BlockSpec((_NUM_BOXES_PADDED, 4), lambda i: (0, 0)),
            pl.BlockSpec((1, 2), lambda i: (0, 0)),
        ],
        out_specs=pl.BlockSpec((1, 1), lambda i: (0, 0)),
        out_shape=jax.ShapeDtypeStruct((1, 1), jnp.float32),
        scratch_shapes=[pltpu.VMEM((1, 128), jnp.float32)],
        compiler_params=pltpu.CompilerParams(
            dimension_semantics=("arbitrary",),
        ),
    )(h0f, h1f, h2f, lab, dims)
    return out.reshape(())


# speedup vs baseline: 1.0973x; 1.0973x over previous
import jax
import jax.numpy as jnp
from jax.experimental import pallas as pl
from jax.experimental.pallas import tpu as pltpu

_NUM_BOXES_PADDED = 128
_LEVELS = ((8, 200, 336), (8, 100, 168), (8, 50, 84))
_STEPS = 8


def _lamm_body(h0_ref, h1_ref, h2_ref, lab_ref, dims_ref, out_ref,
               acc0_ref, acc1_ref, acc2_ref, area_ref):
    i = pl.program_id(0)
    dimx = dims_ref[0, 0]
    dimy = dims_ref[0, 1]
    lab = lab_ref[:, :]

    for h_ref, acc_ref in ((h0_ref, acc0_ref), (h1_ref, acc1_ref),
                           (h2_ref, acc2_ref)):
        blk = h_ref[0, :, :]
        prev = jnp.where(i == 0, jnp.zeros_like(blk), acc_ref[:, :])
        acc_ref[:, :] = prev + blk

    for j, (_, hgt, wid) in enumerate(_LEVELS):
        @pl.when(i == j)
        def _():
            sx = wid / dimx
            sy = hgt / dimy
            x1 = jnp.clip(jnp.round(lab[:, 0:1] * sx), 0.0, wid - 1.0)
            y1 = jnp.clip(jnp.round(lab[:, 1:2] * sy), 0.0, hgt - 1.0)
            x2 = jnp.clip(jnp.round(lab[:, 2:3] * sx), 0.0, float(wid))
            y2 = jnp.clip(jnp.round(lab[:, 3:4] * sy), 0.0, float(hgt))
            valid = ((x2 > x1) & (y2 > y1)).astype(jnp.float32)
            xx = jax.lax.broadcasted_iota(
                jnp.int32, (_NUM_BOXES_PADDED, wid), 1).astype(jnp.float32)
            yy = jax.lax.broadcasted_iota(
                jnp.int32, (_NUM_BOXES_PADDED, hgt), 1).astype(jnp.float32)
            xm = ((xx >= x1) & (xx < x2)).astype(jnp.float32) * valid
            ym = ((yy >= y1) & (yy < y2)).astype(jnp.float32)
            cov = jax.lax.dot_general(
                ym, xm, (((0,), (0,)), ((), ())),
                preferred_element_type=jnp.float32,
            )
            area = jnp.sum((cov > 0.5).astype(jnp.float32))
            area_ref[0:1, j:j + 1] = jnp.reshape(area, (1, 1))

    @pl.when(i == _STEPS - 1)
    def _():
        total = jnp.float32(0.0)
        for j, (acc_ref, (n, hgt, wid)) in enumerate(
                zip((acc0_ref, acc1_ref, acc2_ref), _LEVELS)):
            s = jnp.sum(acc_ref[:, :])
            li = (s / float(n * hgt * wid)
                  - area_ref[0, j] / float(hgt * wid)) ** 2
            total = total + li
        out_ref[:, :] = jnp.reshape(total / 3.0, (1, 1))


def kernel(h0, h1, h2, label, im_dimx, im_dimy):
    h0f = h0.reshape(8, 200, 336)
    h1f = h1.reshape(8, 100, 168)
    h2f = h2.reshape(8, 50, 84)
    lab = jnp.pad(label.astype(jnp.float32),
                  ((0, _NUM_BOXES_PADDED - label.shape[0]), (0, 0)))
    dims = jnp.stack([jnp.asarray(im_dimx, jnp.float32),
                      jnp.asarray(im_dimy, jnp.float32)]).reshape(1, 2)
    out = pl.pallas_call(
        _lamm_body,
        grid=(_STEPS,),
        in_specs=[
            pl.BlockSpec((1, 200, 336), lambda i: (i, 0, 0)),
            pl.BlockSpec((1, 100, 168), lambda i: (i, 0, 0)),
            pl.BlockSpec((1, 50, 84), lambda i: (i, 0, 0)),
            pl.BlockSpec((_NUM_BOXES_PADDED, 4), lambda i: (0, 0)),
            pl.BlockSpec((1, 2), lambda i: (0, 0)),
        ],
        out_specs=pl.BlockSpec((1, 1), lambda i: (0, 0)),
        out_shape=jax.ShapeDtypeStruct((1, 1), jnp.float32),
        scratch_shapes=[
            pltpu.VMEM((200, 336), jnp.float32),
            pltpu.VMEM((100, 168), jnp.float32),
            pltpu.VMEM((50, 84), jnp.float32),
            pltpu.VMEM((1, 128), jnp.float32),
        ],
        compiler_params=pltpu.CompilerParams(
            dimension_semantics=("arbitrary",),
        ),
    )(h0f, h1f, h2f, lab, dims)
    return out.reshape(())


# speedup vs baseline: 1.6238x; 1.4798x over previous
import jax
import jax.numpy as jnp
from jax.experimental import pallas as pl
from jax.experimental.pallas import tpu as pltpu

_NUM_BOXES_PADDED = 128
_LEVELS = ((8, 200, 336), (8, 100, 168), (8, 50, 84))


def _lamm_body(h0_ref, h1_ref, h2_ref, lab_ref, dims_ref, out_ref,
               v0_ref, v1_ref, v2_ref, sem0, sem1, sem2):
    cp0 = pltpu.make_async_copy(h0_ref, v0_ref, sem0)
    cp1 = pltpu.make_async_copy(h1_ref, v1_ref, sem1)
    cp2 = pltpu.make_async_copy(h2_ref, v2_ref, sem2)
    cp0.start()
    cp1.start()
    cp2.start()

    dimx = dims_ref[0, 0]
    dimy = dims_ref[0, 1]
    lab = lab_ref[:, :]

    areas = []
    for _, hgt, wid in _LEVELS:
        sx = wid / dimx
        sy = hgt / dimy
        x1 = jnp.clip(jnp.round(lab[:, 0:1] * sx), 0.0, wid - 1.0)
        y1 = jnp.clip(jnp.round(lab[:, 1:2] * sy), 0.0, hgt - 1.0)
        x2 = jnp.clip(jnp.round(lab[:, 2:3] * sx), 0.0, float(wid))
        y2 = jnp.clip(jnp.round(lab[:, 3:4] * sy), 0.0, float(hgt))
        valid = ((x2 > x1) & (y2 > y1)).astype(jnp.float32)
        xx = jax.lax.broadcasted_iota(
            jnp.int32, (_NUM_BOXES_PADDED, wid), 1).astype(jnp.float32)
        yy = jax.lax.broadcasted_iota(
            jnp.int32, (_NUM_BOXES_PADDED, hgt), 1).astype(jnp.float32)
        xm = ((xx >= x1) & (xx < x2)).astype(jnp.float32) * valid
        ym = ((yy >= y1) & (yy < y2)).astype(jnp.float32)
        cov = jax.lax.dot_general(
            ym, xm, (((0,), (0,)), ((), ())),
            preferred_element_type=jnp.float32,
        )
        areas.append(jnp.sum((cov > 0.5).astype(jnp.float32)))

    total = jnp.float32(0.0)
    for cp, v_ref, area, (n, hgt, wid) in zip(
            (cp0, cp1, cp2), (v0_ref, v1_ref, v2_ref), areas, _LEVELS):
        cp.wait()
        s = jnp.sum(v_ref[:, :])
        li = (s / float(n * hgt * wid) - area / float(hgt * wid)) ** 2
        total = total + li

    out_ref[:, :] = jnp.reshape(total / 3.0, (1, 1))


def kernel(h0, h1, h2, label, im_dimx, im_dimy):
    h0f = h0.reshape(8 * 200, 336)
    h1f = h1.reshape(8 * 100, 168)
    h2f = h2.reshape(8 * 50, 84)
    lab = jnp.pad(label.astype(jnp.float32),
                  ((0, _NUM_BOXES_PADDED - label.shape[0]), (0, 0)))
    dims = jnp.stack([jnp.asarray(im_dimx, jnp.float32),
                      jnp.asarray(im_dimy, jnp.float32)]).reshape(1, 2)
    out = pl.pallas_call(
        _lamm_body,
        in_specs=[
            pl.BlockSpec(memory_space=pl.ANY),
            pl.BlockSpec(memory_space=pl.ANY),
            pl.BlockSpec(memory_space=pl.ANY),
            pl.BlockSpec(memory_space=pltpu.MemorySpace.VMEM),
            pl.BlockSpec(memory_space=pltpu.MemorySpace.VMEM),
        ],
        out_shape=jax.ShapeDtypeStruct((1, 1), jnp.float32),
        scratch_shapes=[
            pltpu.VMEM((8 * 200, 336), jnp.float32),
            pltpu.VMEM((8 * 100, 168), jnp.float32),
            pltpu.VMEM((8 * 50, 84), jnp.float32),
            pltpu.SemaphoreType.DMA,
            pltpu.SemaphoreType.DMA,
            pltpu.SemaphoreType.DMA,
        ],
    )(h0f, h1f, h2f, lab, dims)
    return out.reshape(())


# speedup vs baseline: 1.9250x; 1.1855x over previous
import jax
import jax.numpy as jnp
from jax.experimental import pallas as pl
from jax.experimental.pallas import tpu as pltpu

_NUM_BOXES = 100
_LEVELS = ((8, 200, 336), (8, 100, 168), (8, 50, 84))


def _lamm_body(h0_ref, h1_ref, h2_ref, lab_ref, dx_ref, dy_ref, out_ref):
    dimx = dx_ref[0, 0]
    dimy = dy_ref[0, 0]
    lab = lab_ref[:, :]

    total = jnp.float32(0.0)
    for h_ref, (n, hgt, wid) in zip((h0_ref, h1_ref, h2_ref), _LEVELS):
        sx = wid / dimx
        sy = hgt / dimy
        x1 = jnp.clip(jnp.round(lab[:, 0:1] * sx), 0.0, wid - 1.0)
        y1 = jnp.clip(jnp.round(lab[:, 1:2] * sy), 0.0, hgt - 1.0)
        x2 = jnp.clip(jnp.round(lab[:, 2:3] * sx), 0.0, float(wid))
        y2 = jnp.clip(jnp.round(lab[:, 3:4] * sy), 0.0, float(hgt))
        valid = ((x2 > x1) & (y2 > y1)).astype(jnp.float32)
        xx = jax.lax.broadcasted_iota(
            jnp.int32, (_NUM_BOXES, wid), 1).astype(jnp.float32)
        yy = jax.lax.broadcasted_iota(
            jnp.int32, (_NUM_BOXES, hgt), 1).astype(jnp.float32)
        xm = (((xx >= x1) & (xx < x2)).astype(jnp.float32)
              * valid).astype(jnp.bfloat16)
        ym = ((yy >= y1) & (yy < y2)).astype(jnp.bfloat16)
        cov = jax.lax.dot_general(
            ym, xm, (((0,), (0,)), ((), ())),
            preferred_element_type=jnp.float32,
        )
        area = jnp.sum((cov > 0.5).astype(jnp.float32))
        s = jnp.sum(h_ref[:, :])
        li = (s / float(n * hgt * wid) - area / float(hgt * wid)) ** 2
        total = total + li

    out_ref[:, :] = jnp.reshape(total / 3.0, (1, 1))


def kernel(h0, h1, h2, label, im_dimx, im_dimy):
    h0f = h0.reshape(8 * 200, 336)
    h1f = h1.reshape(8 * 100, 168)
    h2f = h2.reshape(8 * 50, 84)
    dx = jnp.asarray(im_dimx, jnp.float32).reshape(1, 1)
    dy = jnp.asarray(im_dimy, jnp.float32).reshape(1, 1)
    out = pl.pallas_call(
        _lamm_body,
        in_specs=[
            pl.BlockSpec(memory_space=pltpu.MemorySpace.VMEM),
            pl.BlockSpec(memory_space=pltpu.MemorySpace.VMEM),
            pl.BlockSpec(memory_space=pltpu.MemorySpace.VMEM),
            pl.BlockSpec(memory_space=pltpu.MemorySpace.VMEM),
            pl.BlockSpec(memory_space=pltpu.MemorySpace.SMEM),
            pl.BlockSpec(memory_space=pltpu.MemorySpace.SMEM),
        ],
        out_shape=jax.ShapeDtypeStruct((1, 1), jnp.float32),
    )(h0f, h1f, h2f, label, dx, dy)
    return out.reshape(())
